# Initial kernel scaffold; baseline (speedup 1.0000x reference)
import jax
import jax.numpy as jnp
from jax.experimental import pallas as pl


def kernel(hidden_states, embedding, output_tokens, presence_penalties, frequency_penalties, repetition_penalties, top_ps, top_ks):
    raise NotImplementedError("write your pallas kernel here")


# speedup vs baseline: 13.7453x; 13.7453x over previous
import functools

import jax
import jax.numpy as jnp
from jax import lax
from jax.experimental import pallas as pl
from jax.experimental.pallas import tpu as pltpu
from jax.experimental.pallas import tpu_sc as plsc

B = 64
V = 100000
D = 1024
L = 50
TOKP = 64
CHUNK = 2048
NCHUNK = (V + CHUNK - 1) // CHUNK
STATC = 64
NEG = -1e30
NVREG = V // 16
CAP1 = 20480
CAP2 = 2048


def _logits_body(hs_ref, emb_ref, out_ref, cmax_ref, csum_ref):
    c = pl.program_id(0)
    l = lax.dot_general(
        hs_ref[...], emb_ref[...], (((1,), (1,)), ((), ())),
        preferred_element_type=jnp.float32,
        precision=lax.Precision.HIGHEST,
    )
    col = c * CHUNK + lax.broadcasted_iota(jnp.int32, (B, CHUNK), 1)
    l = jnp.where(col < V, l, NEG)
    out_ref[...] = l
    m = jnp.max(l, axis=1)
    cmax_ref[...] = m.reshape(1, 1, B)
    csum_ref[...] = jnp.sum(jnp.exp(l - m[:, None]), axis=1).reshape(1, 1, B)


def _tc_logits(hs, emb):
    return pl.pallas_call(
        _logits_body,
        grid=(NCHUNK,),
        in_specs=[
            pl.BlockSpec((B, D), lambda c: (0, 0)),
            pl.BlockSpec((CHUNK, D), lambda c: (c, 0)),
        ],
        out_specs=[
            pl.BlockSpec((B, CHUNK), lambda c: (0, c)),
            pl.BlockSpec((1, 1, B), lambda c: (c, 0, 0)),
            pl.BlockSpec((1, 1, B), lambda c: (c, 0, 0)),
        ],
        out_shape=[
            jax.ShapeDtypeStruct((B, V), jnp.float32),
            jax.ShapeDtypeStruct((STATC, 1, B), jnp.float32),
            jax.ShapeDtypeStruct((STATC, 1, B), jnp.float32),
        ],
        compiler_params=pltpu.CompilerParams(
            dimension_semantics=("arbitrary",)),
    )(hs, emb)


def _iota16():
    return lax.broadcasted_iota(jnp.int32, (16,), 0)


def _key(x):
    b = plsc.bitcast(x, jnp.int32)
    s = lax.shift_right_arithmetic(b, 31)
    k = b ^ (s | jnp.int32(-2147483648))
    return plsc.bitcast(k, jnp.uint32)


def _pick_f(buf_ref, r):
    acc = jnp.float32(0.0)
    for q in range(4):
        v = buf_ref[pl.ds(q * 16, 16)]
        m = (_iota16() + q * 16) == r
        acc = acc + jnp.sum(jnp.where(m, v, 0.0))
    return acc


def _pick_i(buf_ref, r):
    acc = jnp.int32(0)
    for q in range(4):
        v = buf_ref[pl.ds(q * 16, 16)]
        m = (_iota16() + q * 16) == r
        acc = acc + jnp.sum(jnp.where(m, v, 0))
    return acc


def _scalar(x):
    return jnp.max(x) if getattr(x, "ndim", 0) else x


def _sel_from_groups(groups, b):
    zero = jnp.zeros((16,), groups[0].dtype)
    acc = None
    for g, vec in enumerate(groups):
        m = (_iota16() + g * 16) == b
        part = jnp.sum(jnp.where(m, vec, zero))
        acc = part if acc is None else acc + part
    return acc


def _level_scan(totc, tote, A, E, topk, toppZ):
    nb = len(totc) * 16
    runc = jnp.int32(0)
    rune = jnp.float32(0.0)
    inclc, incle = [], []
    for g in range(len(totc)):
        inclc.append(plsc.cumsum(totc[g]) + runc)
        incle.append(plsc.cumsum(tote[g]) + rune)
        runc = runc + jnp.sum(totc[g])
        rune = rune + jnp.sum(tote[g])
    RT, ERT = runc, rune
    nfalse = jnp.int32(0)
    for g in range(len(totc)):
        cnt_ge = A + (RT - inclc[g]) + totc[g]
        e_ge = E + (ERT - incle[g]) + tote[g]
        kn = (cnt_ge < topk) & (e_ge <= toppZ)
        nfalse = nfalse + jnp.sum(jnp.where(kn, 0, 1))
    bstar = nfalse - 1
    cnt_b = _sel_from_groups(totc, bstar)
    inclc_b = _sel_from_groups(inclc, bstar)
    incle_b = _sel_from_groups(incle, bstar)
    above_b = A + (RT - inclc_b)
    eabove_b = E + (ERT - incle_b)
    descend = (above_b < topk) & (eabove_b <= toppZ)
    return bstar, above_b, eabove_b, cnt_b, descend


def _clear_hists(histc_v, histe_v, nwords):
    zi = jnp.zeros((16,), jnp.int32)
    zf = jnp.zeros((16,), jnp.float32)

    def body(j, carry):
        histc_v[pl.ds(j * 16, 16)] = zi
        histe_v[pl.ds(j * 16, 16)] = zf
        return carry

    lax.fori_loop(0, nwords // 16, body, 0)


def _hist_totals(histc_v, histe_v, nb):
    totc, tote = [], []
    for g in range(nb // 16):
        ac = jnp.zeros((16,), jnp.int32)
        ae = jnp.zeros((16,), jnp.float32)
        for lane in range(16):
            ac = ac + histc_v[pl.ds(lane * nb + g * 16, 16)]
            ae = ae + histe_v[pl.ds(lane * nb + g * 16, 16)]
        totc.append(ac)
        tote.append(ae)
    return totc, tote


def _sc_body(logits_hbm, tok_hbm, pres_hbm, freq_hbm, rep_hbm, topp_hbm,
             topk_hbm, cmax_hbm, csum_hbm, out_hbm,
             row_v, cand1_v, cand2_v, histc_v, histe_v, tok_v,
             cmax_v, csum_v, pres_v, freq_v, rep_v, topp_v, topk_v):
    cid = lax.axis_index("c")
    sid = lax.axis_index("s")
    w = sid * 2 + cid

    pltpu.sync_copy(pres_hbm, pres_v)
    pltpu.sync_copy(freq_hbm, freq_v)
    pltpu.sync_copy(rep_hbm, rep_v)
    pltpu.sync_copy(topp_hbm, topp_v)
    pltpu.sync_copy(topk_hbm, topk_v)

    iota = _iota16()
    lane_b1 = iota * 128
    lane_b2 = iota * 32
    ones_i = jnp.ones((16,), jnp.int32)

    def do_row(i, carry):
        r = w * 2 + i
        pltpu.sync_copy(logits_hbm.at[r], row_v)
        pltpu.sync_copy(tok_hbm.at[r], tok_v)
        pltpu.sync_copy(cmax_hbm.at[r], cmax_v)
        pltpu.sync_copy(csum_hbm.at[r], csum_v)

        pres_r = _pick_f(pres_v, r)
        freq_r = _pick_f(freq_v, r)
        rep_r = _pick_f(rep_v, r)
        topp_r = _pick_f(topp_v, r)
        topk_r = _pick_i(topk_v, r)

        m_acc = jnp.full((16,), NEG, jnp.float32)
        for q in range(4):
            gid = iota + q * 16
            cm = jnp.where(gid < NCHUNK, cmax_v[pl.ds(q * 16, 16)], NEG)
            m_acc = jnp.maximum(m_acc, cm)
        M = jnp.max(m_acc)
        z_acc = jnp.zeros((16,), jnp.float32)
        for q in range(4):
            gid = iota + q * 16
            ok = gid < NCHUNK
            cm = jnp.where(ok, cmax_v[pl.ds(q * 16, 16)], NEG)
            cs = jnp.where(ok, csum_v[pl.ds(q * 16, 16)], 0.0)
            z_acc = z_acc + jnp.where(ok, cs * jnp.exp(cm - M), 0.0)
        Z = jnp.sum(z_acc)

        tvs = [tok_v[pl.ds(q * 16, 16)] for q in range(4)]

        def cnt_body(p, cnts):
            c0, c1, c2, c3 = cnts
            tp = jnp.int32(0)
            for q in range(4):
                tp = tp + jnp.sum(jnp.where((iota + q * 16) == p, tvs[q], 0))
            upd = []
            for q, cq in enumerate((c0, c1, c2, c3)):
                upd.append(cq + (tvs[q] == tp).astype(jnp.int32))
            return tuple(upd)

        zcnt = jnp.zeros((16,), jnp.int32)
        cnts = lax.fori_loop(0, TOKP, cnt_body, (zcnt, zcnt, zcnt, zcnt))

        valids = [t < V for t in tvs]
        idxs = [jnp.where(vld, t, 0) for t, vld in zip(tvs, valids)]
        olds = [plsc.load_gather(row_v, [ix], mask=vld)
                for ix, vld in zip(idxs, valids)]
        news = []
        corr = jnp.float32(0.0)
        for q in range(4):
            old = olds[q]
            cntf = cnts[q].astype(jnp.float32)
            nv = jnp.where(old > 0, old / rep_r, old * rep_r)
            nv = nv - (freq_r * cntf + pres_r)
            news.append(nv)
            safe_cnt = jnp.maximum(cntf, 1.0)
            dq = (jnp.exp(old - M) - jnp.exp(nv - M)) / safe_cnt
            corr = corr + jnp.sum(jnp.where(valids[q], dq, 0.0))
        for q in range(4):
            plsc.store_scatter(row_v, [idxs[q]], news[q], mask=valids[q])
        Z = Z - corr
        toppZ = topp_r * Z

        _clear_hists(histc_v, histe_v, 2048)

        def h1(j, carry):
            v = row_v[pl.ds(j * 16, 16)]
            u = _key(v)
            d = lax.shift_right_logical(u, jnp.uint32(25)).astype(jnp.int32)
            idx = lane_b1 + d
            plsc.addupdate_scatter(histc_v, [idx], ones_i)
            plsc.addupdate_scatter(histe_v, [idx], jnp.exp(v - M))
            return carry

        lax.fori_loop(0, NVREG, h1, 0)
        totc, tote = _hist_totals(histc_v, histe_v, 128)
        A = jnp.int32(0)
        E = jnp.float32(0.0)
        b1, a1, e1, c1, desc1 = _level_scan(totc, tote, A, E, topk_r, toppZ)
        done = jnp.logical_not(desc1)
        theta = jnp.where(
            done, (b1 + 1).astype(jnp.uint32) << jnp.uint32(25), jnp.uint32(0))
        A = jnp.where(desc1, a1, A)
        E = jnp.where(desc1, e1, E)
        prefix = jnp.where(desc1, b1.astype(jnp.uint32) << jnp.uint32(25),
                           jnp.uint32(0))
        b1_eff = jnp.where(desc1, b1, jnp.int32(-1))

        def comp1(j, off):
            v = row_v[pl.ds(j * 16, 16)]
            u = _key(v)
            d = lax.shift_right_logical(u, jnp.uint32(25)).astype(jnp.int32)
            m = d == b1_eff
            o = jnp.minimum(off, CAP1 - 16)
            plsc.store_compressed(cand1_v.at[pl.ds(o, 16)], v, mask=m)
            return off + _scalar(plsc.all_reduce_population_count(m))

        n1 = lax.fori_loop(0, NVREG, comp1, jnp.int32(0))
        n1 = jnp.minimum(n1, CAP1)

        _clear_hists(histc_v, histe_v, 512)

        def h2(j, carry):
            v = cand1_v[pl.ds(j * 16, 16)]
            u = _key(v)
            d = (lax.shift_right_logical(u, jnp.uint32(20))
                 & jnp.uint32(31)).astype(jnp.int32)
            m = (j * 16 + iota) < n1
            idx = lane_b2 + d
            plsc.addupdate_scatter(histc_v, [idx], ones_i, mask=m)
            plsc.addupdate_scatter(histe_v, [idx], jnp.exp(v - M), mask=m)
            return carry

        lax.fori_loop(0, (n1 + 15) // 16, h2, 0)
        totc, tote = _hist_totals(histc_v, histe_v, 32)
        b2, a2, e2, c2, desc2 = _level_scan(totc, tote, A, E, topk_r, toppZ)
        newly_done = jnp.logical_not(done) & jnp.logical_not(desc2)
        theta = jnp.where(
            newly_done,
            prefix + ((b2 + 1).astype(jnp.uint32) << jnp.uint32(20)), theta)
        go = jnp.logical_not(done) & desc2
        A = jnp.where(go, a2, A)
        E = jnp.where(go, e2, E)
        prefix = jnp.where(go, prefix | (b2.astype(jnp.uint32)
                                         << jnp.uint32(20)), prefix)
        done = done | newly_done
        b2_eff = jnp.where(go, b2, jnp.int32(-1))

        def comp2(j, off):
            v = cand1_v[pl.ds(j * 16, 16)]
            u = _key(v)
            d = (lax.shift_right_logical(u, jnp.uint32(20))
                 & jnp.uint32(31)).astype(jnp.int32)
            m = (d == b2_eff) & ((j * 16 + iota) < n1)
            o = jnp.minimum(off, CAP2 - 16)
            plsc.store_compressed(cand2_v.at[pl.ds(o, 16)], v, mask=m)
            return off + _scalar(plsc.all_reduce_population_count(m))

        n2 = lax.fori_loop(0, (n1 + 15) // 16, comp2, jnp.int32(0))
        n2 = jnp.minimum(n2, CAP2)

        for shift in (15, 10, 5, 0):
            _clear_hists(histc_v, histe_v, 512)
            himask = jnp.uint32((0xFFFFFFFF << (shift + 5)) & 0xFFFFFFFF)

            def hk(j, carry, _s=shift, _hm=himask):
                v = cand2_v[pl.ds(j * 16, 16)]
                u = _key(v)
                d = (lax.shift_right_logical(u, jnp.uint32(_s))
                     & jnp.uint32(31)).astype(jnp.int32)
                m = (((u ^ prefix) & _hm) == jnp.uint32(0)) \
                    & ((j * 16 + iota) < n2)
                idx = lane_b2 + d
                plsc.addupdate_scatter(histc_v, [idx], ones_i, mask=m)
                plsc.addupdate_scatter(histe_v, [idx], jnp.exp(v - M), mask=m)
                return carry

            lax.fori_loop(0, (n2 + 15) // 16, hk, 0)
            totc, tote = _hist_totals(histc_v, histe_v, 32)
            bk, ak, ek, ck, desck = _level_scan(totc, tote, A, E,
                                                topk_r, toppZ)
            if shift == 0:
                final_theta = prefix | bk.astype(jnp.uint32)
                theta = jnp.where(done, theta, final_theta)
            else:
                newly_done = jnp.logical_not(done) & jnp.logical_not(desck)
                theta = jnp.where(
                    newly_done,
                    prefix + ((bk + 1).astype(jnp.uint32)
                              << jnp.uint32(shift)), theta)
                go = jnp.logical_not(done) & desck
                A = jnp.where(go, ak, A)
                E = jnp.where(go, ek, E)
                prefix = jnp.where(go, prefix | (bk.astype(jnp.uint32)
                                                 << jnp.uint32(shift)),
                                   prefix)
                done = done | newly_done

        ninf = jnp.float32(-jnp.inf)

        def mpass(j, carry):
            v = row_v[pl.ds(j * 16, 16)]
            u = _key(v)
            row_v[pl.ds(j * 16, 16)] = jnp.where(u >= theta, v, ninf)
            return carry

        lax.fori_loop(0, NVREG, mpass, 0)
        pltpu.sync_copy(row_v, out_hbm.at[r])
        return carry

    lax.fori_loop(0, 2, do_row, 0)


_SC_CACHE = []


def _sc_sampler(*args):
    if not _SC_CACHE:
        _SC_CACHE.append(functools.partial(
            pl.kernel,
            out_type=jax.ShapeDtypeStruct((B, V), jnp.float32),
            mesh=plsc.VectorSubcoreMesh(core_axis_name="c",
                                        subcore_axis_name="s",
                                        num_cores=2, num_subcores=16),
            compiler_params=pltpu.CompilerParams(needs_layout_passes=False),
            scratch_types=[
                pltpu.VMEM((V,), jnp.float32),
                pltpu.VMEM((CAP1,), jnp.float32),
                pltpu.VMEM((CAP2,), jnp.float32),
                pltpu.VMEM((2048,), jnp.int32),
                pltpu.VMEM((2048,), jnp.float32),
                pltpu.VMEM((TOKP,), jnp.int32),
                pltpu.VMEM((STATC,), jnp.float32),
                pltpu.VMEM((STATC,), jnp.float32),
                pltpu.VMEM((B,), jnp.float32),
                pltpu.VMEM((B,), jnp.float32),
                pltpu.VMEM((B,), jnp.float32),
                pltpu.VMEM((B,), jnp.float32),
                pltpu.VMEM((B,), jnp.int32),
            ],
        )(_sc_body))
    return _SC_CACHE[0](*args)


def kernel(hidden_states, embedding, output_tokens, presence_penalties,
           frequency_penalties, repetition_penalties, top_ps, top_ks):
    logits, cmax3, csum3 = _tc_logits(hidden_states, embedding)
    cmax = cmax3[:, 0, :].T
    csum = csum3[:, 0, :].T
    tok_pad = jnp.concatenate(
        [output_tokens.astype(jnp.int32),
         jnp.full((B, TOKP - L), V, jnp.int32)], axis=1)
    return _sc_sampler(
        logits, tok_pad, presence_penalties, frequency_penalties,
        repetition_penalties, top_ps, top_ks, cmax, csum)


# speedup vs baseline: 14.2034x; 1.0333x over previous
import functools

import jax
import jax.numpy as jnp
from jax import lax
from jax.experimental import pallas as pl
from jax.experimental.pallas import tpu as pltpu
from jax.experimental.pallas import tpu_sc as plsc

B = 64
V = 100000
D = 1024
L = 50
TOKP = 64
CHUNK = 2048
NCHUNK = (V + CHUNK - 1) // CHUNK
STATC = 64
NEG = -1e30
NVREG = V // 16
CAP1 = 20480
CAP2 = 2048


def _logits_body(hs_ref, emb_ref, out_ref, cmax_ref, csum_ref):
    c = pl.program_id(0)
    l = lax.dot_general(
        hs_ref[...], emb_ref[...], (((1,), (1,)), ((), ())),
        preferred_element_type=jnp.float32,
        precision=lax.Precision.HIGHEST,
    )
    col = c * CHUNK + lax.broadcasted_iota(jnp.int32, (B, CHUNK), 1)
    l = jnp.where(col < V, l, NEG)
    out_ref[...] = l
    m = jnp.max(l, axis=1)
    cmax_ref[...] = m.reshape(1, 1, B)
    csum_ref[...] = jnp.sum(jnp.exp(l - m[:, None]), axis=1).reshape(1, 1, B)


def _tc_logits(hs, emb):
    return pl.pallas_call(
        _logits_body,
        grid=(NCHUNK,),
        in_specs=[
            pl.BlockSpec((B, D), lambda c: (0, 0)),
            pl.BlockSpec((CHUNK, D), lambda c: (c, 0)),
        ],
        out_specs=[
            pl.BlockSpec((B, CHUNK), lambda c: (0, c)),
            pl.BlockSpec((1, 1, B), lambda c: (c, 0, 0)),
            pl.BlockSpec((1, 1, B), lambda c: (c, 0, 0)),
        ],
        out_shape=[
            jax.ShapeDtypeStruct((B, V), jnp.float32),
            jax.ShapeDtypeStruct((STATC, 1, B), jnp.float32),
            jax.ShapeDtypeStruct((STATC, 1, B), jnp.float32),
        ],
        compiler_params=pltpu.CompilerParams(
            dimension_semantics=("arbitrary",)),
    )(hs, emb)


def _iota16():
    return lax.broadcasted_iota(jnp.int32, (16,), 0)


def _key(x):
    b = plsc.bitcast(x, jnp.int32)
    s = lax.shift_right_arithmetic(b, 31)
    k = b ^ (s | jnp.int32(-2147483648))
    return plsc.bitcast(k, jnp.uint32)


def _pick_f(buf_ref, r):
    acc = jnp.float32(0.0)
    for q in range(4):
        v = buf_ref[pl.ds(q * 16, 16)]
        m = (_iota16() + q * 16) == r
        acc = acc + jnp.sum(jnp.where(m, v, 0.0))
    return acc


def _pick_i(buf_ref, r):
    acc = jnp.int32(0)
    for q in range(4):
        v = buf_ref[pl.ds(q * 16, 16)]
        m = (_iota16() + q * 16) == r
        acc = acc + jnp.sum(jnp.where(m, v, 0))
    return acc


def _scalar(x):
    return jnp.max(x) if getattr(x, "ndim", 0) else x


def _sel_from_groups(groups, b):
    zero = jnp.zeros((16,), groups[0].dtype)
    acc = None
    for g, vec in enumerate(groups):
        m = (_iota16() + g * 16) == b
        part = jnp.sum(jnp.where(m, vec, zero))
        acc = part if acc is None else acc + part
    return acc


def _level_scan(totc, tote, A, E, topk, toppZ):
    nb = len(totc) * 16
    runc = jnp.int32(0)
    rune = jnp.float32(0.0)
    inclc, incle = [], []
    for g in range(len(totc)):
        inclc.append(plsc.cumsum(totc[g]) + runc)
        incle.append(plsc.cumsum(tote[g]) + rune)
        runc = runc + jnp.sum(totc[g])
        rune = rune + jnp.sum(tote[g])
    RT, ERT = runc, rune
    nfalse = jnp.int32(0)
    for g in range(len(totc)):
        cnt_ge = A + (RT - inclc[g]) + totc[g]
        e_ge = E + (ERT - incle[g]) + tote[g]
        kn = (cnt_ge < topk) & (e_ge <= toppZ)
        nfalse = nfalse + jnp.sum(jnp.where(kn, 0, 1))
    bstar = nfalse - 1
    cnt_b = _sel_from_groups(totc, bstar)
    inclc_b = _sel_from_groups(inclc, bstar)
    incle_b = _sel_from_groups(incle, bstar)
    above_b = A + (RT - inclc_b)
    eabove_b = E + (ERT - incle_b)
    descend = (above_b < topk) & (eabove_b <= toppZ)
    return bstar, above_b, eabove_b, cnt_b, descend


def _clear_hists(histc_v, histe_v, nwords):
    zi = jnp.zeros((16,), jnp.int32)
    zf = jnp.zeros((16,), jnp.float32)

    def body(j, carry):
        histc_v[pl.ds(j * 16, 16)] = zi
        histe_v[pl.ds(j * 16, 16)] = zf
        return carry

    lax.fori_loop(0, nwords // 16, body, 0, unroll=8)


def _hist_totals(histc_v, histe_v, nb):
    totc, tote = [], []
    for g in range(nb // 16):
        ac = jnp.zeros((16,), jnp.int32)
        ae = jnp.zeros((16,), jnp.float32)
        for lane in range(16):
            ac = ac + histc_v[pl.ds(lane * nb + g * 16, 16)]
            ae = ae + histe_v[pl.ds(lane * nb + g * 16, 16)]
        totc.append(ac)
        tote.append(ae)
    return totc, tote


def _sc_body(logits_hbm, tok_hbm, pres_hbm, freq_hbm, rep_hbm, topp_hbm,
             topk_hbm, cmax_hbm, csum_hbm, out_hbm,
             row_v, cand1_v, cand2_v, histc_v, histe_v, tok_v,
             cmax_v, csum_v, pres_v, freq_v, rep_v, topp_v, topk_v):
    cid = lax.axis_index("c")
    sid = lax.axis_index("s")
    w = sid * 2 + cid

    pltpu.sync_copy(pres_hbm, pres_v)
    pltpu.sync_copy(freq_hbm, freq_v)
    pltpu.sync_copy(rep_hbm, rep_v)
    pltpu.sync_copy(topp_hbm, topp_v)
    pltpu.sync_copy(topk_hbm, topk_v)

    iota = _iota16()
    lane_b1 = iota * 128
    lane_b2 = iota * 32
    ones_i = jnp.ones((16,), jnp.int32)

    def do_row(i, carry):
        r = w * 2 + i
        pltpu.sync_copy(logits_hbm.at[r], row_v)
        pltpu.sync_copy(tok_hbm.at[r], tok_v)
        pltpu.sync_copy(cmax_hbm.at[r], cmax_v)
        pltpu.sync_copy(csum_hbm.at[r], csum_v)

        pres_r = _pick_f(pres_v, r)
        freq_r = _pick_f(freq_v, r)
        rep_r = _pick_f(rep_v, r)
        topp_r = _pick_f(topp_v, r)
        topk_r = _pick_i(topk_v, r)

        m_acc = jnp.full((16,), NEG, jnp.float32)
        for q in range(4):
            gid = iota + q * 16
            cm = jnp.where(gid < NCHUNK, cmax_v[pl.ds(q * 16, 16)], NEG)
            m_acc = jnp.maximum(m_acc, cm)
        M = jnp.max(m_acc)
        z_acc = jnp.zeros((16,), jnp.float32)
        for q in range(4):
            gid = iota + q * 16
            ok = gid < NCHUNK
            cm = jnp.where(ok, cmax_v[pl.ds(q * 16, 16)], NEG)
            cs = jnp.where(ok, csum_v[pl.ds(q * 16, 16)], 0.0)
            z_acc = z_acc + jnp.where(ok, cs * jnp.exp(cm - M), 0.0)
        Z = jnp.sum(z_acc)

        tvs = [tok_v[pl.ds(q * 16, 16)] for q in range(4)]

        def cnt_body(p, cnts):
            c0, c1, c2, c3 = cnts
            tp = jnp.int32(0)
            for q in range(4):
                tp = tp + jnp.sum(jnp.where((iota + q * 16) == p, tvs[q], 0))
            upd = []
            for q, cq in enumerate((c0, c1, c2, c3)):
                upd.append(cq + (tvs[q] == tp).astype(jnp.int32))
            return tuple(upd)

        zcnt = jnp.zeros((16,), jnp.int32)
        cnts = lax.fori_loop(0, TOKP, cnt_body, (zcnt, zcnt, zcnt, zcnt))

        valids = [t < V for t in tvs]
        idxs = [jnp.where(vld, t, 0) for t, vld in zip(tvs, valids)]
        olds = [plsc.load_gather(row_v, [ix], mask=vld)
                for ix, vld in zip(idxs, valids)]
        news = []
        corr = jnp.float32(0.0)
        for q in range(4):
            old = olds[q]
            cntf = cnts[q].astype(jnp.float32)
            nv = jnp.where(old > 0, old / rep_r, old * rep_r)
            nv = nv - (freq_r * cntf + pres_r)
            news.append(nv)
            safe_cnt = jnp.maximum(cntf, 1.0)
            dq = (jnp.exp(old - M) - jnp.exp(nv - M)) / safe_cnt
            corr = corr + jnp.sum(jnp.where(valids[q], dq, 0.0))
        for q in range(4):
            plsc.store_scatter(row_v, [idxs[q]], news[q], mask=valids[q])
        Z = Z - corr
        toppZ = topp_r * Z

        _clear_hists(histc_v, histe_v, 2048)

        def h1(j, carry):
            v = row_v[pl.ds(j * 16, 16)]
            u = _key(v)
            d = lax.shift_right_logical(u, jnp.uint32(25)).astype(jnp.int32)
            idx = lane_b1 + d
            plsc.addupdate_scatter(histc_v, [idx], ones_i)
            plsc.addupdate_scatter(histe_v, [idx], jnp.exp(v - M))
            return carry

        lax.fori_loop(0, NVREG, h1, 0, unroll=10)
        totc, tote = _hist_totals(histc_v, histe_v, 128)
        A = jnp.int32(0)
        E = jnp.float32(0.0)
        b1, a1, e1, c1, desc1 = _level_scan(totc, tote, A, E, topk_r, toppZ)
        done = jnp.logical_not(desc1)
        theta = jnp.where(
            done, (b1 + 1).astype(jnp.uint32) << jnp.uint32(25), jnp.uint32(0))
        A = jnp.where(desc1, a1, A)
        E = jnp.where(desc1, e1, E)
        prefix = jnp.where(desc1, b1.astype(jnp.uint32) << jnp.uint32(25),
                           jnp.uint32(0))
        b1_eff = jnp.where(desc1, b1, jnp.int32(-1))

        def comp1(j, off):
            v = row_v[pl.ds(j * 16, 16)]
            u = _key(v)
            d = lax.shift_right_logical(u, jnp.uint32(25)).astype(jnp.int32)
            m = d == b1_eff
            o = jnp.minimum(off, CAP1 - 16)
            plsc.store_compressed(cand1_v.at[pl.ds(o, 16)], v, mask=m)
            return off + _scalar(plsc.all_reduce_population_count(m))

        n1 = lax.fori_loop(0, NVREG, comp1, jnp.int32(0), unroll=10)
        n1 = jnp.minimum(n1, CAP1)

        _clear_hists(histc_v, histe_v, 512)

        def h2(j, carry):
            v = cand1_v[pl.ds(j * 16, 16)]
            u = _key(v)
            d = (lax.shift_right_logical(u, jnp.uint32(20))
                 & jnp.uint32(31)).astype(jnp.int32)
            m = (j * 16 + iota) < n1
            idx = lane_b2 + d
            plsc.addupdate_scatter(histc_v, [idx], ones_i, mask=m)
            plsc.addupdate_scatter(histe_v, [idx], jnp.exp(v - M), mask=m)
            return carry

        lax.fori_loop(0, (n1 + 15) // 16, h2, 0)
        totc, tote = _hist_totals(histc_v, histe_v, 32)
        b2, a2, e2, c2, desc2 = _level_scan(totc, tote, A, E, topk_r, toppZ)
        newly_done = jnp.logical_not(done) & jnp.logical_not(desc2)
        theta = jnp.where(
            newly_done,
            prefix + ((b2 + 1).astype(jnp.uint32) << jnp.uint32(20)), theta)
        go = jnp.logical_not(done) & desc2
        A = jnp.where(go, a2, A)
        E = jnp.where(go, e2, E)
        prefix = jnp.where(go, prefix | (b2.astype(jnp.uint32)
                                         << jnp.uint32(20)), prefix)
        done = done | newly_done
        b2_eff = jnp.where(go, b2, jnp.int32(-1))

        def comp2(j, off):
            v = cand1_v[pl.ds(j * 16, 16)]
            u = _key(v)
            d = (lax.shift_right_logical(u, jnp.uint32(20))
                 & jnp.uint32(31)).astype(jnp.int32)
            m = (d == b2_eff) & ((j * 16 + iota) < n1)
            o = jnp.minimum(off, CAP2 - 16)
            plsc.store_compressed(cand2_v.at[pl.ds(o, 16)], v, mask=m)
            return off + _scalar(plsc.all_reduce_population_count(m))

        n2 = lax.fori_loop(0, (n1 + 15) // 16, comp2, jnp.int32(0))
        n2 = jnp.minimum(n2, CAP2)

        for shift in (15, 10, 5, 0):
            _clear_hists(histc_v, histe_v, 512)
            himask = jnp.uint32((0xFFFFFFFF << (shift + 5)) & 0xFFFFFFFF)

            def hk(j, carry, _s=shift, _hm=himask):
                v = cand2_v[pl.ds(j * 16, 16)]
                u = _key(v)
                d = (lax.shift_right_logical(u, jnp.uint32(_s))
                     & jnp.uint32(31)).astype(jnp.int32)
                m = (((u ^ prefix) & _hm) == jnp.uint32(0)) \
                    & ((j * 16 + iota) < n2)
                idx = lane_b2 + d
                plsc.addupdate_scatter(histc_v, [idx], ones_i, mask=m)
                plsc.addupdate_scatter(histe_v, [idx], jnp.exp(v - M), mask=m)
                return carry

            lax.fori_loop(0, (n2 + 15) // 16, hk, 0)
            totc, tote = _hist_totals(histc_v, histe_v, 32)
            bk, ak, ek, ck, desck = _level_scan(totc, tote, A, E,
                                                topk_r, toppZ)
            if shift == 0:
                final_theta = prefix | bk.astype(jnp.uint32)
                theta = jnp.where(done, theta, final_theta)
            else:
                newly_done = jnp.logical_not(done) & jnp.logical_not(desck)
                theta = jnp.where(
                    newly_done,
                    prefix + ((bk + 1).astype(jnp.uint32)
                              << jnp.uint32(shift)), theta)
                go = jnp.logical_not(done) & desck
                A = jnp.where(go, ak, A)
                E = jnp.where(go, ek, E)
                prefix = jnp.where(go, prefix | (bk.astype(jnp.uint32)
                                                 << jnp.uint32(shift)),
                                   prefix)
                done = done | newly_done

        ninf = jnp.float32(-jnp.inf)

        def mpass(j, carry):
            v = row_v[pl.ds(j * 16, 16)]
            u = _key(v)
            row_v[pl.ds(j * 16, 16)] = jnp.where(u >= theta, v, ninf)
            return carry

        lax.fori_loop(0, NVREG, mpass, 0, unroll=10)
        pltpu.sync_copy(row_v, out_hbm.at[r])
        return carry

    lax.fori_loop(0, 2, do_row, 0)


_SC_CACHE = []


def _sc_sampler(*args):
    if not _SC_CACHE:
        _SC_CACHE.append(functools.partial(
            pl.kernel,
            out_type=jax.ShapeDtypeStruct((B, V), jnp.float32),
            mesh=plsc.VectorSubcoreMesh(core_axis_name="c",
                                        subcore_axis_name="s",
                                        num_cores=2, num_subcores=16),
            compiler_params=pltpu.CompilerParams(needs_layout_passes=False),
            scratch_types=[
                pltpu.VMEM((V,), jnp.float32),
                pltpu.VMEM((CAP1,), jnp.float32),
                pltpu.VMEM((CAP2,), jnp.float32),
                pltpu.VMEM((2048,), jnp.int32),
                pltpu.VMEM((2048,), jnp.float32),
                pltpu.VMEM((TOKP,), jnp.int32),
                pltpu.VMEM((STATC,), jnp.float32),
                pltpu.VMEM((STATC,), jnp.float32),
                pltpu.VMEM((B,), jnp.float32),
                pltpu.VMEM((B,), jnp.float32),
                pltpu.VMEM((B,), jnp.float32),
                pltpu.VMEM((B,), jnp.float32),
                pltpu.VMEM((B,), jnp.int32),
            ],
        )(_sc_body))
    return _SC_CACHE[0](*args)


def kernel(hidden_states, embedding, output_tokens, presence_penalties,
           frequency_penalties, repetition_penalties, top_ps, top_ks):
    logits, cmax3, csum3 = _tc_logits(hidden_states, embedding)
    cmax = cmax3[:, 0, :].T
    csum = csum3[:, 0, :].T
    tok_pad = jnp.concatenate(
        [output_tokens.astype(jnp.int32),
         jnp.full((B, TOKP - L), V, jnp.int32)], axis=1)
    return _sc_sampler(
        logits, tok_pad, presence_penalties, frequency_penalties,
        repetition_penalties, top_ps, top_ks, cmax, csum)


# speedup vs baseline: 16.1485x; 1.1369x over previous
import functools

import jax
import jax.numpy as jnp
from jax import lax
from jax.experimental import pallas as pl
from jax.experimental.pallas import tpu as pltpu
from jax.experimental.pallas import tpu_sc as plsc

B = 64
V = 100000
D = 1024
L = 50
TOKP = 64
CHUNK = 2048
NCHUNK = (V + CHUNK - 1) // CHUNK
STATC = 64
NEG = -1e30
NVREG = V // 16
CAP1 = 12288
CAP2 = 6144


def _logits_body(hs_ref, emb_ref, out_ref, cmax_ref, csum_ref):
    c = pl.program_id(0)
    l = lax.dot_general(
        hs_ref[...], emb_ref[...], (((1,), (1,)), ((), ())),
        preferred_element_type=jnp.float32,
        precision=lax.Precision.HIGHEST,
    )
    col = c * CHUNK + lax.broadcasted_iota(jnp.int32, (B, CHUNK), 1)
    l = jnp.where(col < V, l, NEG)
    out_ref[...] = l
    m = jnp.max(l, axis=1)
    cmax_ref[...] = m.reshape(1, 1, B)
    csum_ref[...] = jnp.sum(jnp.exp(l - m[:, None]), axis=1).reshape(1, 1, B)


def _tc_logits(hs, emb):
    return pl.pallas_call(
        _logits_body,
        grid=(NCHUNK,),
        in_specs=[
            pl.BlockSpec((B, D), lambda c: (0, 0)),
            pl.BlockSpec((CHUNK, D), lambda c: (c, 0)),
        ],
        out_specs=[
            pl.BlockSpec((B, CHUNK), lambda c: (0, c)),
            pl.BlockSpec((1, 1, B), lambda c: (c, 0, 0)),
            pl.BlockSpec((1, 1, B), lambda c: (c, 0, 0)),
        ],
        out_shape=[
            jax.ShapeDtypeStruct((B, V), jnp.float32),
            jax.ShapeDtypeStruct((STATC, 1, B), jnp.float32),
            jax.ShapeDtypeStruct((STATC, 1, B), jnp.float32),
        ],
        compiler_params=pltpu.CompilerParams(
            dimension_semantics=("arbitrary",)),
    )(hs, emb)


def _iota16():
    return lax.broadcasted_iota(jnp.int32, (16,), 0)


def _key(x):
    b = plsc.bitcast(x, jnp.int32)
    s = lax.shift_right_arithmetic(b, 31)
    k = b ^ (s | jnp.int32(-2147483648))
    return plsc.bitcast(k, jnp.uint32)


def _pick_f(buf_ref, r):
    acc = jnp.float32(0.0)
    for q in range(4):
        v = buf_ref[pl.ds(q * 16, 16)]
        m = (_iota16() + q * 16) == r
        acc = acc + jnp.sum(jnp.where(m, v, 0.0))
    return acc


def _pick_i(buf_ref, r):
    acc = jnp.int32(0)
    for q in range(4):
        v = buf_ref[pl.ds(q * 16, 16)]
        m = (_iota16() + q * 16) == r
        acc = acc + jnp.sum(jnp.where(m, v, 0))
    return acc


def _scalar(x):
    return jnp.max(x) if getattr(x, "ndim", 0) else x


def _sel_from_groups(groups, b):
    zero = jnp.zeros((16,), groups[0].dtype)
    acc = None
    for g, vec in enumerate(groups):
        m = (_iota16() + g * 16) == b
        part = jnp.sum(jnp.where(m, vec, zero))
        acc = part if acc is None else acc + part
    return acc


def _level_scan(totc, tote, A, E, topk, toppZ):
    nb = len(totc) * 16
    runc = jnp.int32(0)
    rune = jnp.float32(0.0)
    inclc, incle = [], []
    for g in range(len(totc)):
        inclc.append(plsc.cumsum(totc[g]) + runc)
        incle.append(plsc.cumsum(tote[g]) + rune)
        runc = runc + jnp.sum(totc[g])
        rune = rune + jnp.sum(tote[g])
    RT, ERT = runc, rune
    nfalse = jnp.int32(0)
    for g in range(len(totc)):
        cnt_ge = A + (RT - inclc[g]) + totc[g]
        e_ge = E + (ERT - incle[g]) + tote[g]
        kn = (cnt_ge < topk) & (e_ge <= toppZ)
        nfalse = nfalse + jnp.sum(jnp.where(kn, 0, 1))
    bstar = nfalse - 1
    cnt_b = _sel_from_groups(totc, bstar)
    inclc_b = _sel_from_groups(inclc, bstar)
    incle_b = _sel_from_groups(incle, bstar)
    above_b = A + (RT - inclc_b)
    eabove_b = E + (ERT - incle_b)
    descend = (above_b < topk) & (eabove_b <= toppZ)
    return bstar, above_b, eabove_b, cnt_b, descend


def _clear_hists(histc_v, histe_v, nwords):
    zi = jnp.zeros((16,), jnp.int32)
    zf = jnp.zeros((16,), jnp.float32)

    def body(j, carry):
        histc_v[pl.ds(j * 16, 16)] = zi
        if histe_v is not None:
            histe_v[pl.ds(j * 16, 16)] = zf
        return carry

    lax.fori_loop(0, nwords // 16, body, 0, unroll=8)


def _hist_idx(d):
    return (d << 4) | ((_iota16() + d) & 15)


def _hist_totals(hist_ref, nb):
    out = []
    zero = jnp.zeros((16,), hist_ref.dtype)
    for g in range(nb // 16):
        acc = zero
        for j in range(16):
            s = jnp.sum(hist_ref[pl.ds((g * 16 + j) * 16, 16)])
            acc = jnp.where(_iota16() == j, s, acc)
        out.append(acc)
    return out


def _sc_body(logits_hbm, tok_hbm, pres_hbm, freq_hbm, rep_hbm, topp_hbm,
             topk_hbm, cmax_hbm, csum_hbm, out_hbm,
             row_v, cand1_v, cand2_v, histc_v, histe_v, tok_v,
             cmax_v, csum_v, pres_v, freq_v, rep_v, topp_v, topk_v):
    cid = lax.axis_index("c")
    sid = lax.axis_index("s")
    w = sid * 2 + cid

    pltpu.sync_copy(pres_hbm, pres_v)
    pltpu.sync_copy(freq_hbm, freq_v)
    pltpu.sync_copy(rep_hbm, rep_v)
    pltpu.sync_copy(topp_hbm, topp_v)
    pltpu.sync_copy(topk_hbm, topk_v)

    iota = _iota16()
    lane_b1 = iota * 128
    lane_b2 = iota * 32
    ones_i = jnp.ones((16,), jnp.int32)

    def do_row(i, carry):
        r = w * 2 + i
        pltpu.sync_copy(logits_hbm.at[r], row_v)
        pltpu.sync_copy(tok_hbm.at[r], tok_v)
        pltpu.sync_copy(cmax_hbm.at[r], cmax_v)
        pltpu.sync_copy(csum_hbm.at[r], csum_v)

        pres_r = _pick_f(pres_v, r)
        freq_r = _pick_f(freq_v, r)
        rep_r = _pick_f(rep_v, r)
        topp_r = _pick_f(topp_v, r)
        topk_r = _pick_i(topk_v, r)

        m_acc = jnp.full((16,), NEG, jnp.float32)
        for q in range(4):
            gid = iota + q * 16
            cm = jnp.where(gid < NCHUNK, cmax_v[pl.ds(q * 16, 16)], NEG)
            m_acc = jnp.maximum(m_acc, cm)
        M = jnp.max(m_acc)
        z_acc = jnp.zeros((16,), jnp.float32)
        for q in range(4):
            gid = iota + q * 16
            ok = gid < NCHUNK
            cm = jnp.where(ok, cmax_v[pl.ds(q * 16, 16)], NEG)
            cs = jnp.where(ok, csum_v[pl.ds(q * 16, 16)], 0.0)
            z_acc = z_acc + jnp.where(ok, cs * jnp.exp(cm - M), 0.0)
        Z = jnp.sum(z_acc)

        tvs = [tok_v[pl.ds(q * 16, 16)] for q in range(4)]

        def cnt_body(p, cnts):
            c0, c1, c2, c3 = cnts
            tp = jnp.int32(0)
            for q in range(4):
                tp = tp + jnp.sum(jnp.where((iota + q * 16) == p, tvs[q], 0))
            upd = []
            for q, cq in enumerate((c0, c1, c2, c3)):
                upd.append(cq + (tvs[q] == tp).astype(jnp.int32))
            return tuple(upd)

        zcnt = jnp.zeros((16,), jnp.int32)
        cnts = lax.fori_loop(0, TOKP, cnt_body, (zcnt, zcnt, zcnt, zcnt))

        valids = [t < V for t in tvs]
        idxs = [jnp.where(vld, t, 0) for t, vld in zip(tvs, valids)]
        olds = [plsc.load_gather(row_v, [ix], mask=vld)
                for ix, vld in zip(idxs, valids)]
        news = []
        corr = jnp.float32(0.0)
        for q in range(4):
            old = olds[q]
            cntf = cnts[q].astype(jnp.float32)
            nv = jnp.where(old > 0, old / rep_r, old * rep_r)
            nv = nv - (freq_r * cntf + pres_r)
            news.append(nv)
            safe_cnt = jnp.maximum(cntf, 1.0)
            dq = (jnp.exp(old - M) - jnp.exp(nv - M)) / safe_cnt
            corr = corr + jnp.sum(jnp.where(valids[q], dq, 0.0))
        for q in range(4):
            plsc.store_scatter(row_v, [idxs[q]], news[q], mask=valids[q])
        Z = Z - corr
        toppZ = topp_r * Z

        _clear_hists(histc_v, None, 2048)

        def ha(j, carry):
            v = row_v[pl.ds(j * 16, 16)]
            u = _key(v)
            d = lax.shift_right_logical(u, jnp.uint32(25)).astype(jnp.int32)
            plsc.addupdate_scatter(histc_v, [_hist_idx(d)], ones_i)
            return carry

        lax.fori_loop(0, NVREG, ha, 0, unroll=10)
        totc = _hist_totals(histc_v, 128)
        runc = jnp.int32(0)
        inclc = []
        for g in range(8):
            inclc.append(plsc.cumsum(totc[g]) + runc)
            runc = runc + jnp.sum(totc[g])
        RT = runc
        nfalse = jnp.int32(0)
        for g in range(8):
            kn = (RT - inclc[g] + totc[g]) < topk_r
            nfalse = nfalse + jnp.sum(jnp.where(kn, 0, 1))
        bk0 = nfalse - 1

        def compA(j, off):
            v = row_v[pl.ds(j * 16, 16)]
            u = _key(v)
            d = lax.shift_right_logical(u, jnp.uint32(25)).astype(jnp.int32)
            m = d >= bk0
            o = jnp.minimum(off, CAP1 - 16)
            plsc.store_compressed(cand1_v.at[pl.ds(o, 16)], v, mask=m)
            return off + _scalar(plsc.all_reduce_population_count(m))

        n1 = lax.fori_loop(0, NVREG, compA, jnp.int32(0), unroll=10)
        n1 = jnp.minimum(n1, CAP1)

        _clear_hists(histc_v, histe_v, 2048)

        def h1(j, carry):
            v = cand1_v[pl.ds(j * 16, 16)]
            u = _key(v)
            d = lax.shift_right_logical(u, jnp.uint32(25)).astype(jnp.int32)
            m = (j * 16 + iota) < n1
            idx = _hist_idx(d)
            plsc.addupdate_scatter(histc_v, [idx], ones_i, mask=m)
            plsc.addupdate_scatter(histe_v, [idx], jnp.exp(v - M), mask=m)
            return carry

        lax.fori_loop(0, (n1 + 15) // 16, h1, 0)
        totc = _hist_totals(histc_v, 128)
        tote = _hist_totals(histe_v, 128)
        A = jnp.int32(0)
        E = jnp.float32(0.0)
        b1, a1, e1, c1, desc1 = _level_scan(totc, tote, A, E, topk_r, toppZ)
        done = jnp.logical_not(desc1)
        theta = jnp.where(
            done, (b1 + 1).astype(jnp.uint32) << jnp.uint32(25), jnp.uint32(0))
        A = jnp.where(desc1, a1, A)
        E = jnp.where(desc1, e1, E)
        prefix = jnp.where(desc1, b1.astype(jnp.uint32) << jnp.uint32(25),
                           jnp.uint32(0))
        b1_eff = jnp.where(desc1, b1, jnp.int32(-1))

        def comp1(j, off):
            v = cand1_v[pl.ds(j * 16, 16)]
            u = _key(v)
            d = lax.shift_right_logical(u, jnp.uint32(25)).astype(jnp.int32)
            m = (d == b1_eff) & ((j * 16 + iota) < n1)
            o = jnp.minimum(off, CAP2 - 16)
            plsc.store_compressed(cand2_v.at[pl.ds(o, 16)], v, mask=m)
            return off + _scalar(plsc.all_reduce_population_count(m))

        n2 = lax.fori_loop(0, (n1 + 15) // 16, comp1, jnp.int32(0))
        n2 = jnp.minimum(n2, CAP2)

        _clear_hists(histc_v, histe_v, 512)

        def h2(j, carry):
            v = cand2_v[pl.ds(j * 16, 16)]
            u = _key(v)
            d = (lax.shift_right_logical(u, jnp.uint32(20))
                 & jnp.uint32(31)).astype(jnp.int32)
            m = (j * 16 + iota) < n2
            idx = _hist_idx(d)
            plsc.addupdate_scatter(histc_v, [idx], ones_i, mask=m)
            plsc.addupdate_scatter(histe_v, [idx], jnp.exp(v - M), mask=m)
            return carry

        lax.fori_loop(0, (n2 + 15) // 16, h2, 0)
        totc = _hist_totals(histc_v, 32)
        tote = _hist_totals(histe_v, 32)
        b2, a2, e2, c2, desc2 = _level_scan(totc, tote, A, E, topk_r, toppZ)
        newly_done = jnp.logical_not(done) & jnp.logical_not(desc2)
        theta = jnp.where(
            newly_done,
            prefix + ((b2 + 1).astype(jnp.uint32) << jnp.uint32(20)), theta)
        go = jnp.logical_not(done) & desc2
        A = jnp.where(go, a2, A)
        E = jnp.where(go, e2, E)
        prefix = jnp.where(go, prefix | (b2.astype(jnp.uint32)
                                         << jnp.uint32(20)), prefix)
        done = done | newly_done
        b2_eff = jnp.where(go, b2, jnp.int32(-1))

        def comp2(j, off):
            v = cand2_v[pl.ds(j * 16, 16)]
            u = _key(v)
            d = (lax.shift_right_logical(u, jnp.uint32(20))
                 & jnp.uint32(31)).astype(jnp.int32)
            m = (d == b2_eff) & ((j * 16 + iota) < n2)
            o = jnp.minimum(off, CAP1 - 16)
            plsc.store_compressed(cand1_v.at[pl.ds(o, 16)], v, mask=m)
            return off + _scalar(plsc.all_reduce_population_count(m))

        n3 = lax.fori_loop(0, (n2 + 15) // 16, comp2, jnp.int32(0))
        n3 = jnp.minimum(n3, CAP1)

        for shift in (15, 10, 5, 0):
            _clear_hists(histc_v, histe_v, 512)
            himask = jnp.uint32((0xFFFFFFFF << (shift + 5)) & 0xFFFFFFFF)

            def hk(j, carry, _s=shift, _hm=himask):
                v = cand1_v[pl.ds(j * 16, 16)]
                u = _key(v)
                d = (lax.shift_right_logical(u, jnp.uint32(_s))
                     & jnp.uint32(31)).astype(jnp.int32)
                m = (((u ^ prefix) & _hm) == jnp.uint32(0)) \
                    & ((j * 16 + iota) < n3)
                idx = _hist_idx(d)
                plsc.addupdate_scatter(histc_v, [idx], ones_i, mask=m)
                plsc.addupdate_scatter(histe_v, [idx], jnp.exp(v - M), mask=m)
                return carry

            lax.fori_loop(0, (n3 + 15) // 16, hk, 0)
            totc = _hist_totals(histc_v, 32)
            tote = _hist_totals(histe_v, 32)
            bk, ak, ek, ck, desck = _level_scan(totc, tote, A, E,
                                                topk_r, toppZ)
            if shift == 0:
                final_theta = prefix | bk.astype(jnp.uint32)
                theta = jnp.where(done, theta, final_theta)
            else:
                newly_done = jnp.logical_not(done) & jnp.logical_not(desck)
                theta = jnp.where(
                    newly_done,
                    prefix + ((bk + 1).astype(jnp.uint32)
                              << jnp.uint32(shift)), theta)
                go = jnp.logical_not(done) & desck
                A = jnp.where(go, ak, A)
                E = jnp.where(go, ek, E)
                prefix = jnp.where(go, prefix | (bk.astype(jnp.uint32)
                                                 << jnp.uint32(shift)),
                                   prefix)
                done = done | newly_done

        ninf = jnp.float32(-jnp.inf)

        def mpass(j, carry):
            v = row_v[pl.ds(j * 16, 16)]
            u = _key(v)
            row_v[pl.ds(j * 16, 16)] = jnp.where(u >= theta, v, ninf)
            return carry

        lax.fori_loop(0, NVREG, mpass, 0, unroll=10)
        pltpu.sync_copy(row_v, out_hbm.at[r])
        return carry

    lax.fori_loop(0, 2, do_row, 0)


_SC_CACHE = []


def _sc_sampler(*args):
    if not _SC_CACHE:
        _SC_CACHE.append(functools.partial(
            pl.kernel,
            out_type=jax.ShapeDtypeStruct((B, V), jnp.float32),
            mesh=plsc.VectorSubcoreMesh(core_axis_name="c",
                                        subcore_axis_name="s",
                                        num_cores=2, num_subcores=16),
            compiler_params=pltpu.CompilerParams(needs_layout_passes=False),
            scratch_types=[
                pltpu.VMEM((V,), jnp.float32),
                pltpu.VMEM((CAP1,), jnp.float32),
                pltpu.VMEM((CAP2,), jnp.float32),
                pltpu.VMEM((2048,), jnp.int32),
                pltpu.VMEM((2048,), jnp.float32),
                pltpu.VMEM((TOKP,), jnp.int32),
                pltpu.VMEM((STATC,), jnp.float32),
                pltpu.VMEM((STATC,), jnp.float32),
                pltpu.VMEM((B,), jnp.float32),
                pltpu.VMEM((B,), jnp.float32),
                pltpu.VMEM((B,), jnp.float32),
                pltpu.VMEM((B,), jnp.float32),
                pltpu.VMEM((B,), jnp.int32),
            ],
        )(_sc_body))
    return _SC_CACHE[0](*args)


def kernel(hidden_states, embedding, output_tokens, presence_penalties,
           frequency_penalties, repetition_penalties, top_ps, top_ks):
    logits, cmax3, csum3 = _tc_logits(hidden_states, embedding)
    cmax = cmax3[:, 0, :].T
    csum = csum3[:, 0, :].T
    tok_pad = jnp.concatenate(
        [output_tokens.astype(jnp.int32),
         jnp.full((B, TOKP - L), V, jnp.int32)], axis=1)
    return _sc_sampler(
        logits, tok_pad, presence_penalties, frequency_penalties,
        repetition_penalties, top_ps, top_ks, cmax, csum)
